# baseline (speedup 1.0000x reference)
import functools

import jax
import jax.numpy as jnp
from jax.experimental import pallas as pl
from jax.experimental.pallas import tpu as pltpu

_NUM_ETA = 19


def _body(mask_ref, eta_ref, z_ref, o_ref):
    zb = z_ref[0]
    mask = mask_ref[0]
    shp = mask.shape
    A = jnp.zeros(shp, jnp.float32)
    B = jnp.zeros(shp, jnp.float32)
    C = jnp.zeros(shp, jnp.float32)
    D = jnp.zeros(shp, jnp.float32)
    for t in range(_NUM_ETA):
        sel = mask == t
        e0 = eta_ref[t, 0]
        e1 = eta_ref[t, 1]
        e2 = eta_ref[t, 2]
        e3 = eta_ref[t, 3]
        A = jnp.where(sel, -e0, A)
        B = jnp.where(sel, -e1, B)
        C = jnp.where(sel, e3, C)
        D = jnp.where(sel, e2 * e3, D)
    o_ref[0] = A[None] + B[None] * jnp.tanh(zb * C[None] - D[None])


@functools.partial(jax.jit, static_argnames=("interpret",))
def kernel(z, Mask, eta_table, interpret=False):
    M, N, S, F = z.shape
    NB = 128
    mask3 = Mask.astype(jnp.int32).reshape(M, 1, F)
    out = pl.pallas_call(
        _body,
        grid=(M, N // NB),
        in_specs=[
            pl.BlockSpec((1, 1, F), lambda m, n: (m, 0, 0)),
            pl.BlockSpec(memory_space=pltpu.SMEM),
            pl.BlockSpec((1, NB, S, F), lambda m, n: (m, n, 0, 0)),
        ],
        out_specs=pl.BlockSpec((1, NB, S, F), lambda m, n: (m, n, 0, 0)),
        out_shape=jax.ShapeDtypeStruct((M, N, S, F), jnp.float32),
        interpret=interpret,
    )(mask3, eta_table, z)
    return out


# speedup vs baseline: 3.7251x; 3.7251x over previous
import functools

import jax
import jax.numpy as jnp
from jax.experimental import pallas as pl
from jax.experimental.pallas import tpu as pltpu


def _body(mask_ref, eta_ref, z_ref, o_ref):
    i = pl.program_id(0)
    t = mask_ref[i]
    A = -eta_ref[t, 0]
    B = -eta_ref[t, 1]
    C = eta_ref[t, 3]
    D = eta_ref[t, 2] * C
    o_ref[...] = A + B * jnp.tanh(z_ref[...] * C - D)


@functools.partial(jax.jit, static_argnames=("interpret",))
def kernel(z, Mask, eta_table, interpret=False):
    M, N, S, F = z.shape
    NB = 512
    zt = jnp.transpose(z, (0, 3, 1, 2)).reshape(M * F, N, S)
    mask_flat = Mask.astype(jnp.int32).reshape(M * F)
    out = pl.pallas_call(
        _body,
        grid=(M * F, N // NB),
        in_specs=[
            pl.BlockSpec(memory_space=pltpu.SMEM),
            pl.BlockSpec(memory_space=pltpu.SMEM),
            pl.BlockSpec((1, NB, S), lambda i, n: (i, n, 0)),
        ],
        out_specs=pl.BlockSpec((1, NB, S), lambda i, n: (i, n, 0)),
        out_shape=jax.ShapeDtypeStruct((M * F, N, S), jnp.float32),
        interpret=interpret,
    )(mask_flat, eta_table, zt)
    return out.reshape(M, F, N, S).transpose(0, 2, 3, 1)


# speedup vs baseline: 11.5496x; 3.1005x over previous
import functools

import jax
import jax.numpy as jnp
from jax.experimental import pallas as pl
from jax.experimental.pallas import tpu as pltpu


_IB = 4


def _body(mask_ref, eta_ref, z_ref, o_ref):
    i = pl.program_id(0)
    for j in range(_IB):
        t = mask_ref[i * _IB + j]
        A = -eta_ref[t, 0]
        B = -eta_ref[t, 1]
        C = eta_ref[t, 3]
        D = eta_ref[t, 2] * C
        o_ref[j] = A + B * jnp.tanh(z_ref[j] * C - D)


@functools.partial(jax.jit, static_argnames=("interpret",))
def kernel(z, Mask, eta_table, interpret=False):
    M, N, S, F = z.shape
    zt = jnp.transpose(z, (0, 3, 1, 2)).reshape(M * F, N, S)
    mask_flat = Mask.astype(jnp.int32).reshape(M * F)
    out = pl.pallas_call(
        _body,
        grid=(M * F // _IB,),
        in_specs=[
            pl.BlockSpec(memory_space=pltpu.SMEM),
            pl.BlockSpec(memory_space=pltpu.SMEM),
            pl.BlockSpec((_IB, N, S), lambda i: (i, 0, 0)),
        ],
        out_specs=pl.BlockSpec((_IB, N, S), lambda i: (i, 0, 0)),
        out_shape=jax.ShapeDtypeStruct((M * F, N, S), jnp.float32),
        interpret=interpret,
    )(mask_flat, eta_table, zt)
    return out.reshape(M, F, N, S).transpose(0, 2, 3, 1)


# speedup vs baseline: 12.8031x; 1.1085x over previous
import functools

import jax
import jax.numpy as jnp
from jax.experimental import pallas as pl
from jax.experimental.pallas import tpu as pltpu


_IB = 8


def _body(mask_ref, eta_ref, z_ref, o_ref):
    i = pl.program_id(0)
    for j in range(_IB):
        t = mask_ref[i * _IB + j]
        A = -eta_ref[t, 0]
        B = -eta_ref[t, 1]
        C = eta_ref[t, 3]
        D = eta_ref[t, 2] * C
        o_ref[j] = A + B * jnp.tanh(z_ref[j] * C - D)


@functools.partial(jax.jit, static_argnames=("interpret",))
def kernel(z, Mask, eta_table, interpret=False):
    M, N, S, F = z.shape
    zt = jnp.transpose(z, (0, 3, 1, 2)).reshape(M * F, N, S)
    mask_flat = Mask.astype(jnp.int32).reshape(M * F)
    out = pl.pallas_call(
        _body,
        grid=(M * F // _IB,),
        in_specs=[
            pl.BlockSpec(memory_space=pltpu.SMEM),
            pl.BlockSpec(memory_space=pltpu.SMEM),
            pl.BlockSpec((_IB, N, S), lambda i: (i, 0, 0)),
        ],
        out_specs=pl.BlockSpec((_IB, N, S), lambda i: (i, 0, 0)),
        out_shape=jax.ShapeDtypeStruct((M * F, N, S), jnp.float32),
        interpret=interpret,
    )(mask_flat, eta_table, zt)
    return out.reshape(M, F, N, S).transpose(0, 2, 3, 1)


# speedup vs baseline: 13.0520x; 1.0194x over previous
import functools

import jax
import jax.numpy as jnp
from jax.experimental import pallas as pl
from jax.experimental.pallas import tpu as pltpu


_IB = 13


def _body(mask_ref, eta_ref, z_ref, o_ref):
    i = pl.program_id(0)
    for j in range(_IB):
        t = mask_ref[i * _IB + j]
        A = -eta_ref[t, 0]
        B = -eta_ref[t, 1]
        C = eta_ref[t, 3]
        D = eta_ref[t, 2] * C
        o_ref[j] = A + B * jnp.tanh(z_ref[j] * C - D)


@functools.partial(jax.jit, static_argnames=("interpret",))
def kernel(z, Mask, eta_table, interpret=False):
    M, N, S, F = z.shape
    zt = jnp.transpose(z, (0, 3, 1, 2)).reshape(M * F, N, S)
    mask_flat = Mask.astype(jnp.int32).reshape(M * F)
    out = pl.pallas_call(
        _body,
        grid=(M * F // _IB,),
        in_specs=[
            pl.BlockSpec(memory_space=pltpu.SMEM),
            pl.BlockSpec(memory_space=pltpu.SMEM),
            pl.BlockSpec((_IB, N, S), lambda i: (i, 0, 0)),
        ],
        out_specs=pl.BlockSpec((_IB, N, S), lambda i: (i, 0, 0)),
        out_shape=jax.ShapeDtypeStruct((M * F, N, S), jnp.float32),
        interpret=interpret,
    )(mask_flat, eta_table, zt)
    return out.reshape(M, F, N, S).transpose(0, 2, 3, 1)
